# Initial kernel scaffold; baseline (speedup 1.0000x reference)
import jax
import jax.numpy as jnp
from jax.experimental import pallas as pl


def kernel(hidden_states, W_router, Wg, Wu, Wd):
    raise NotImplementedError("write your pallas kernel here")


# speedup vs baseline: 1.3053x; 1.3053x over previous
import functools

import jax
import jax.numpy as jnp
from jax import lax
from jax.experimental import pallas as pl
from jax.experimental.pallas import tpu as pltpu
from jax.experimental.pallas import tpu_sc as plsc

E = 8
S = 2048
H = 1024
I = 4096
TB = 128
NBLK = 40
PADTOT = NBLK * TB
IT = 4
TI = I // IT

_NEG_INF = float("-inf")


def _router_body(x_ref, wr_ref, d0_ref, d1_ref, w0_ref, w1_ref, meta_ref,
                 aux_ref, oh_ref):
    x = x_ref[...]
    logits = lax.dot_general(x, wr_ref[...], (((1,), (1,)), ((), ())),
                             preferred_element_type=jnp.float32)
    iota8 = lax.broadcasted_iota(jnp.int32, (S, E), 1).astype(jnp.float32)
    v0 = jnp.max(logits, axis=1, keepdims=True)
    i0 = jnp.min(jnp.where(logits == v0, iota8, 8.0), axis=1, keepdims=True)
    oh0 = (iota8 == i0).astype(jnp.float32)
    masked = jnp.where(iota8 == i0, _NEG_INF, logits)
    v1 = jnp.max(masked, axis=1, keepdims=True)
    i1 = jnp.min(jnp.where(masked == v1, iota8, 8.0), axis=1, keepdims=True)
    oh1 = (iota8 == i1).astype(jnp.float32)

    g = jnp.exp(v1 - v0)
    s = 1.0 / (1.0 + g)
    w0_ref[...] = jnp.broadcast_to(s, (S, 16))
    w1_ref[...] = jnp.broadcast_to(g * s, (S, 16))

    p = jnp.exp(logits - v0)
    probs = p / jnp.sum(p, axis=1, keepdims=True)
    probs_mean = jnp.sum(probs, axis=0, keepdims=True) * (1.0 / S)
    counts = jnp.sum(oh0 + oh1, axis=0, keepdims=True)
    usage = counts * (1.0 / (S * 2))
    aux_ref[...] = jnp.sum(probs_mean * usage, axis=1, keepdims=True) * float(E)

    nblk = jnp.floor((counts + (TB - 1)) * (1.0 / TB))
    tri8 = (lax.broadcasted_iota(jnp.int32, (E, E), 0)
            < lax.broadcasted_iota(jnp.int32, (E, E), 1)).astype(jnp.float32)
    blk_excl = lax.dot_general(nblk, tri8, (((1,), (0,)), ((), ())),
                               preferred_element_type=jnp.float32)
    pad_off = blk_excl * float(TB)
    nblocks = jnp.sum(nblk, axis=1, keepdims=True)

    bi = lax.broadcasted_iota(jnp.int32, (NBLK, E), 0).astype(jnp.float32)
    emap = jnp.sum((jnp.broadcast_to(blk_excl, (NBLK, E)) <= bi)
                   .astype(jnp.float32), axis=1, keepdims=True) - 1.0
    meta_ref[pl.ds(0, NBLK), :] = emap.astype(jnp.int32)
    meta_ref[pl.ds(NBLK, 8), :] = jnp.broadcast_to(
        nblocks.astype(jnp.int32), (8, 1))

    oh_ref[:, pl.ds(0, E)] = oh0
    oh_ref[:, pl.ds(E, E)] = oh1
    c0tot = jnp.sum(oh0, axis=0, keepdims=True)
    ltri = (lax.broadcasted_iota(jnp.int32, (TB, TB), 1)
            < lax.broadcasted_iota(jnp.int32, (TB, TB), 0)).astype(jnp.bfloat16)

    def chunk(ch, carry):
        ohc = oh_ref[pl.ds(ch * TB, TB), :]
        csc = lax.dot_general(ltri, ohc.astype(jnp.bfloat16),
                              (((1,), (0,)), ((), ())),
                              preferred_element_type=jnp.float32) + carry
        rank0 = csc[:, 0:E]
        rank1 = csc[:, E:2 * E] + c0tot
        d0 = jnp.sum(ohc[:, 0:E] * (pad_off + rank0), axis=1, keepdims=True)
        d1 = jnp.sum(ohc[:, E:2 * E] * (pad_off + rank1), axis=1, keepdims=True)
        d0_ref[pl.ds(ch * TB, TB), :] = d0.astype(jnp.int32)
        d1_ref[pl.ds(ch * TB, TB), :] = d1.astype(jnp.int32)
        return carry + jnp.sum(ohc, axis=0, keepdims=True)

    lax.fori_loop(0, S // TB, chunk, jnp.zeros((1, 2 * E), jnp.float32))


def _router(x, w_router):
    return pl.pallas_call(
        _router_body,
        out_shape=(
            jax.ShapeDtypeStruct((S, 1), jnp.int32),
            jax.ShapeDtypeStruct((S, 1), jnp.int32),
            jax.ShapeDtypeStruct((S, 16), jnp.float32),
            jax.ShapeDtypeStruct((S, 16), jnp.float32),
            jax.ShapeDtypeStruct((NBLK + 8, 1), jnp.int32),
            jax.ShapeDtypeStruct((1, 1), jnp.float32),
        ),
        scratch_shapes=[pltpu.VMEM((S, 2 * E), jnp.float32)],
    )(x, w_router)


_TOK_PER_W = S // 32


def _sc_dispatch_body(x_hbm, d0_hbm, d1_hbm, y_hbm, i0_v, i1_v, x_v, sem):
    wid = lax.axis_index("s") * 2 + lax.axis_index("c")
    base = wid * _TOK_PER_W
    pltpu.sync_copy(d0_hbm.at[pl.ds(base, _TOK_PER_W)], i0_v)
    pltpu.sync_copy(d1_hbm.at[pl.ds(base, _TOK_PER_W)], i1_v)
    pltpu.sync_copy(x_hbm.at[pl.ds(base, _TOK_PER_W)], x_v)
    pltpu.async_copy(x_v, y_hbm.at[i0_v], sem).wait()
    pltpu.async_copy(x_v, y_hbm.at[i1_v], sem).wait()


def _sc_dispatch(x, d0, d1):
    mesh = plsc.VectorSubcoreMesh(core_axis_name="c", subcore_axis_name="s")
    return pl.kernel(
        _sc_dispatch_body,
        out_type=jax.ShapeDtypeStruct((PADTOT, H), jnp.float32),
        mesh=mesh,
        scratch_types=[
            pltpu.VMEM((_TOK_PER_W,), jnp.int32),
            pltpu.VMEM((_TOK_PER_W,), jnp.int32),
            pltpu.VMEM((_TOK_PER_W, H), jnp.float32),
            pltpu.SemaphoreType.DMA,
        ],
    )(x, d0, d1)


def _ffn_body(meta_ref, xg_ref, wg_ref, wu_ref, wd_ref, out_ref, acc_ref):
    it = pl.program_id(0)
    b = pl.program_id(1)
    nb = meta_ref[NBLK]

    @pl.when(b < nb)
    def _():
        x = xg_ref[...].astype(jnp.bfloat16)
        wg = wg_ref[0].astype(jnp.bfloat16)
        wu = wu_ref[0].astype(jnp.bfloat16)
        wd = wd_ref[0].astype(jnp.bfloat16)
        g = lax.dot_general(x, wg, (((1,), (1,)), ((), ())),
                            preferred_element_type=jnp.float32)
        u = lax.dot_general(x, wu, (((1,), (1,)), ((), ())),
                            preferred_element_type=jnp.float32)
        h = (g * jax.nn.sigmoid(g)) * u
        part = lax.dot_general(h.astype(jnp.bfloat16), wd,
                               (((1,), (1,)), ((), ())),
                               preferred_element_type=jnp.float32)
        sl = pl.ds(b * TB, TB)

        @pl.when(it == 0)
        def _():
            acc_ref[sl, :] = part

        @pl.when(jnp.logical_and(it > 0, it < IT - 1))
        def _():
            acc_ref[sl, :] += part

        @pl.when(it == IT - 1)
        def _():
            out_ref[...] = acc_ref[sl, :] + part


def _ffn(meta, y, wg, wu, wd):
    grid_spec = pltpu.PrefetchScalarGridSpec(
        num_scalar_prefetch=1,
        grid=(IT, NBLK),
        in_specs=[
            pl.BlockSpec((TB, H), lambda it, b, m: (b, 0)),
            pl.BlockSpec((1, TI, H), lambda it, b, m: (m[b], it, 0)),
            pl.BlockSpec((1, TI, H), lambda it, b, m: (m[b], it, 0)),
            pl.BlockSpec((1, H, TI), lambda it, b, m: (m[b], 0, it)),
        ],
        out_specs=pl.BlockSpec(
            (TB, H), lambda it, b, m: (jnp.where(it == IT - 1, b, 0), 0)),
        scratch_shapes=[pltpu.VMEM((PADTOT, H), jnp.float32)],
    )
    return pl.pallas_call(
        _ffn_body,
        grid_spec=grid_spec,
        out_shape=jax.ShapeDtypeStruct((PADTOT, H), jnp.float32),
    )(meta, y, wg, wu, wd)


_CHUNK = 32


def _sc_combine_body(y_hbm, d0_hbm, d1_hbm, w0_hbm, w1_hbm, out_hbm,
                     i0_v, i1_v, w0_v, w1_v, a_v, b_v, sem):
    wid = lax.axis_index("s") * 2 + lax.axis_index("c")
    for half in range(_TOK_PER_W // _CHUNK):
        base = wid * _TOK_PER_W + half * _CHUNK
        pltpu.sync_copy(d0_hbm.at[pl.ds(base, _CHUNK)], i0_v)
        pltpu.sync_copy(d1_hbm.at[pl.ds(base, _CHUNK)], i1_v)
        pltpu.sync_copy(w0_hbm.at[pl.ds(base, _CHUNK)], w0_v)
        pltpu.sync_copy(w1_hbm.at[pl.ds(base, _CHUNK)], w1_v)
        pltpu.async_copy(y_hbm.at[i0_v], a_v, sem).wait()
        pltpu.async_copy(y_hbm.at[i1_v], b_v, sem).wait()

        def row(r, _):
            s0 = w0_v[r, :]
            s1 = w1_v[r, :]

            def col(c, _):
                sl = pl.ds(c * 16, 16)
                a_v[r, sl] = s0 * a_v[r, sl] + s1 * b_v[r, sl]
                return 0

            lax.fori_loop(0, H // 16, col, 0)
            return 0

        lax.fori_loop(0, _CHUNK, row, 0)
        pltpu.sync_copy(a_v, out_hbm.at[pl.ds(base, _CHUNK)])


def _sc_combine(y, d0, d1, w0, w1):
    mesh = plsc.VectorSubcoreMesh(core_axis_name="c", subcore_axis_name="s")
    return pl.kernel(
        _sc_combine_body,
        out_type=jax.ShapeDtypeStruct((S, H), jnp.float32),
        mesh=mesh,
        scratch_types=[
            pltpu.VMEM((_CHUNK,), jnp.int32),
            pltpu.VMEM((_CHUNK,), jnp.int32),
            pltpu.VMEM((_CHUNK, 16), jnp.float32),
            pltpu.VMEM((_CHUNK, 16), jnp.float32),
            pltpu.VMEM((_CHUNK, H), jnp.float32),
            pltpu.VMEM((_CHUNK, H), jnp.float32),
            pltpu.SemaphoreType.DMA,
        ],
    )(y, d0, d1, w0, w1)


def kernel(hidden_states, W_router, Wg, Wu, Wd):
    x = hidden_states.reshape(S, H)
    d0c, d1c, w0c, w1c, metac, auxc = _router(x, W_router)
    d0 = d0c.reshape(S)
    d1 = d1c.reshape(S)
    meta = metac.reshape(NBLK + 8)
    y = _sc_dispatch(x, d0, d1)
    z = _ffn(meta, y, Wg, Wu, Wd)
    out = _sc_combine(z, d0, d1, w0c, w1c)
    return out.reshape(1, S, H), auxc.reshape(())


# speedup vs baseline: 1.4609x; 1.1193x over previous
import functools

import jax
import jax.numpy as jnp
from jax import lax
from jax.experimental import pallas as pl
from jax.experimental.pallas import tpu as pltpu
from jax.experimental.pallas import tpu_sc as plsc

E = 8
S = 2048
H = 1024
I = 4096
TB = 128
NBLK = 40
PADTOT = NBLK * TB
IT = 4
TI = I // IT

_NEG_INF = float("-inf")


def _router_body(x_ref, wr_ref, d0_ref, d1_ref, w0_ref, w1_ref, meta_ref,
                 aux_ref, oh_ref):
    x = x_ref[...]
    logits = lax.dot_general(x, wr_ref[...], (((1,), (1,)), ((), ())),
                             preferred_element_type=jnp.float32)
    iota8 = lax.broadcasted_iota(jnp.int32, (S, E), 1).astype(jnp.float32)
    v0 = jnp.max(logits, axis=1, keepdims=True)
    i0 = jnp.min(jnp.where(logits == v0, iota8, 8.0), axis=1, keepdims=True)
    oh0 = (iota8 == i0).astype(jnp.float32)
    masked = jnp.where(iota8 == i0, _NEG_INF, logits)
    v1 = jnp.max(masked, axis=1, keepdims=True)
    i1 = jnp.min(jnp.where(masked == v1, iota8, 8.0), axis=1, keepdims=True)
    oh1 = (iota8 == i1).astype(jnp.float32)

    g = jnp.exp(v1 - v0)
    s = 1.0 / (1.0 + g)
    w0_ref[...] = jnp.broadcast_to(s, (S, 16))
    w1_ref[...] = jnp.broadcast_to(g * s, (S, 16))

    p = jnp.exp(logits - v0)
    probs = p / jnp.sum(p, axis=1, keepdims=True)
    probs_mean = jnp.sum(probs, axis=0, keepdims=True) * (1.0 / S)
    counts = jnp.sum(oh0 + oh1, axis=0, keepdims=True)
    usage = counts * (1.0 / (S * 2))
    aux_ref[...] = jnp.sum(probs_mean * usage, axis=1, keepdims=True) * float(E)

    nblk = jnp.floor((counts + (TB - 1)) * (1.0 / TB))
    tri8 = (lax.broadcasted_iota(jnp.int32, (E, E), 0)
            < lax.broadcasted_iota(jnp.int32, (E, E), 1)).astype(jnp.float32)
    blk_excl = lax.dot_general(nblk, tri8, (((1,), (0,)), ((), ())),
                               preferred_element_type=jnp.float32)
    pad_off = blk_excl * float(TB)
    nblocks = jnp.sum(nblk, axis=1, keepdims=True)

    meta_ref[:, pl.ds(0, E)] = nblk.astype(jnp.int32)
    meta_ref[:, pl.ds(E, E)] = blk_excl.astype(jnp.int32)

    oh_ref[:, pl.ds(0, E)] = oh0
    oh_ref[:, pl.ds(E, E)] = oh1
    c0tot = jnp.sum(oh0, axis=0, keepdims=True)
    ltri = (lax.broadcasted_iota(jnp.int32, (TB, TB), 1)
            < lax.broadcasted_iota(jnp.int32, (TB, TB), 0)).astype(jnp.bfloat16)

    def chunk(ch, carry):
        ohc = oh_ref[pl.ds(ch * TB, TB), :]
        csc = lax.dot_general(ltri, ohc.astype(jnp.bfloat16),
                              (((1,), (0,)), ((), ())),
                              preferred_element_type=jnp.float32) + carry
        rank0 = csc[:, 0:E]
        rank1 = csc[:, E:2 * E] + c0tot
        d0 = jnp.sum(ohc[:, 0:E] * (pad_off + rank0), axis=1, keepdims=True)
        d1 = jnp.sum(ohc[:, E:2 * E] * (pad_off + rank1), axis=1, keepdims=True)
        d0_ref[pl.ds(ch * TB, TB), :] = d0.astype(jnp.int32)
        d1_ref[pl.ds(ch * TB, TB), :] = d1.astype(jnp.int32)
        return carry + jnp.sum(ohc, axis=0, keepdims=True)

    lax.fori_loop(0, S // TB, chunk, jnp.zeros((1, 2 * E), jnp.float32))


def _router(x, w_router):
    return pl.pallas_call(
        _router_body,
        out_shape=(
            jax.ShapeDtypeStruct((S, 1), jnp.int32),
            jax.ShapeDtypeStruct((S, 1), jnp.int32),
            jax.ShapeDtypeStruct((S, 16), jnp.float32),
            jax.ShapeDtypeStruct((S, 16), jnp.float32),
            jax.ShapeDtypeStruct((1, 2 * E), jnp.int32),
            jax.ShapeDtypeStruct((1, 1), jnp.float32),
        ),
        scratch_shapes=[pltpu.VMEM((S, 2 * E), jnp.float32)],
    )(x, w_router)


_TOK_PER_W = S // 32


def _sc_dispatch_body(x_hbm, d0_hbm, d1_hbm, y_hbm, i0_v, i1_v, x_v, sem):
    wid = lax.axis_index("s") * 2 + lax.axis_index("c")
    base = wid * _TOK_PER_W
    pltpu.sync_copy(d0_hbm.at[pl.ds(base, _TOK_PER_W)], i0_v)
    pltpu.sync_copy(d1_hbm.at[pl.ds(base, _TOK_PER_W)], i1_v)
    pltpu.sync_copy(x_hbm.at[pl.ds(base, _TOK_PER_W)], x_v)
    pltpu.async_copy(x_v, y_hbm.at[i0_v], sem).wait()
    pltpu.async_copy(x_v, y_hbm.at[i1_v], sem).wait()


def _sc_dispatch(x, d0, d1):
    mesh = plsc.VectorSubcoreMesh(core_axis_name="c", subcore_axis_name="s")
    return pl.kernel(
        _sc_dispatch_body,
        out_type=jax.ShapeDtypeStruct((PADTOT, H), jnp.float32),
        mesh=mesh,
        scratch_types=[
            pltpu.VMEM((_TOK_PER_W,), jnp.int32),
            pltpu.VMEM((_TOK_PER_W,), jnp.int32),
            pltpu.VMEM((_TOK_PER_W, H), jnp.float32),
            pltpu.SemaphoreType.DMA,
        ],
    )(x, d0, d1)


_MAX_ROWS = 2048


def _ffn_body(meta_ref, xg_hbm, wg_ref, wu_ref, wd_ref, z_hbm,
              xbf_ref, acc_ref, inbuf_ref, outbuf_ref, insem, outsem):
    e = pl.program_id(0)
    it = pl.program_id(1)
    nb = meta_ref[e]
    roff = meta_ref[E + e] * TB

    wgb = wg_ref[0].astype(jnp.bfloat16)
    wub = wu_ref[0].astype(jnp.bfloat16)
    wdb = wd_ref[0].astype(jnp.bfloat16)

    def start_in(j, slot):
        pltpu.make_async_copy(
            xg_hbm.at[pl.ds(roff + j * TB, TB), :],
            inbuf_ref.at[slot], insem.at[slot]).start()

    def wait_in(slot):
        pltpu.make_async_copy(
            xg_hbm.at[pl.ds(0, TB), :],
            inbuf_ref.at[slot], insem.at[slot]).wait()

    def wait_out(slot):
        pltpu.make_async_copy(
            outbuf_ref.at[slot],
            z_hbm.at[pl.ds(0, TB), :], outsem.at[slot]).wait()

    def compute(j, xc):
        g = lax.dot_general(xc, wgb, (((1,), (1,)), ((), ())),
                            preferred_element_type=jnp.float32)
        u = lax.dot_general(xc, wub, (((1,), (1,)), ((), ())),
                            preferred_element_type=jnp.float32)
        h = (g * jax.nn.sigmoid(g)) * u
        part = lax.dot_general(h.astype(jnp.bfloat16), wdb,
                               (((1,), (1,)), ((), ())),
                               preferred_element_type=jnp.float32)
        sl = pl.ds(j * TB, TB)

        @pl.when(it == 0)
        def _():
            acc_ref[sl, :] = part

        @pl.when(jnp.logical_and(it > 0, it < IT - 1))
        def _():
            acc_ref[sl, :] += part

        @pl.when(it == IT - 1)
        def _():
            res = acc_ref[sl, :] + part
            slot = lax.rem(j, 2)

            @pl.when(j >= 2)
            def _():
                wait_out(slot)

            outbuf_ref[slot] = res
            pltpu.make_async_copy(
                outbuf_ref.at[slot],
                z_hbm.at[pl.ds(roff + j * TB, TB), :],
                outsem.at[slot]).start()

    @pl.when(it == 0)
    def _():
        @pl.when(nb > 0)
        def _():
            start_in(0, 0)

        @pl.when(nb > 1)
        def _():
            start_in(1, 1)

        def body0(j, c):
            slot = lax.rem(j, 2)
            wait_in(slot)
            xc = inbuf_ref[slot].astype(jnp.bfloat16)
            xbf_ref[pl.ds(j * TB, TB), :] = xc

            @pl.when(j + 2 < nb)
            def _():
                start_in(j + 2, slot)

            compute(j, xc)
            return c

        lax.fori_loop(0, nb, body0, 0)

    @pl.when(it != 0)
    def _():
        def bodyn(j, c):
            compute(j, xbf_ref[pl.ds(j * TB, TB), :])
            return c

        lax.fori_loop(0, nb, bodyn, 0)

    @pl.when(it == IT - 1)
    def _():
        @pl.when(nb >= 2)
        def _():
            wait_out(lax.rem(nb, 2))

        @pl.when(nb >= 1)
        def _():
            wait_out(lax.rem(nb - 1, 2))


def _ffn(meta, y, wg, wu, wd):
    grid_spec = pltpu.PrefetchScalarGridSpec(
        num_scalar_prefetch=1,
        grid=(E, IT),
        in_specs=[
            pl.BlockSpec(memory_space=pltpu.HBM),
            pl.BlockSpec((1, TI, H), lambda e, it, m: (e, it, 0)),
            pl.BlockSpec((1, TI, H), lambda e, it, m: (e, it, 0)),
            pl.BlockSpec((1, H, TI), lambda e, it, m: (e, 0, it)),
        ],
        out_specs=pl.BlockSpec(memory_space=pltpu.HBM),
        scratch_shapes=[
            pltpu.VMEM((_MAX_ROWS, H), jnp.bfloat16),
            pltpu.VMEM((_MAX_ROWS, H), jnp.float32),
            pltpu.VMEM((2, TB, H), jnp.float32),
            pltpu.VMEM((2, TB, H), jnp.float32),
            pltpu.SemaphoreType.DMA((2,)),
            pltpu.SemaphoreType.DMA((2,)),
        ],
    )
    return pl.pallas_call(
        _ffn_body,
        grid_spec=grid_spec,
        out_shape=jax.ShapeDtypeStruct((PADTOT, H), jnp.float32),
    )(meta, y, wg, wu, wd)


_CHUNK = 32


def _sc_combine_body(y_hbm, d0_hbm, d1_hbm, w0_hbm, w1_hbm, out_hbm,
                     i0_v, i1_v, w0_v, w1_v, a_v, b_v, sem):
    wid = lax.axis_index("s") * 2 + lax.axis_index("c")
    for half in range(_TOK_PER_W // _CHUNK):
        base = wid * _TOK_PER_W + half * _CHUNK
        pltpu.sync_copy(d0_hbm.at[pl.ds(base, _CHUNK)], i0_v)
        pltpu.sync_copy(d1_hbm.at[pl.ds(base, _CHUNK)], i1_v)
        pltpu.sync_copy(w0_hbm.at[pl.ds(base, _CHUNK)], w0_v)
        pltpu.sync_copy(w1_hbm.at[pl.ds(base, _CHUNK)], w1_v)
        pltpu.async_copy(y_hbm.at[i0_v], a_v, sem).wait()
        pltpu.async_copy(y_hbm.at[i1_v], b_v, sem).wait()

        def row(r, _):
            s0 = w0_v[r, :]
            s1 = w1_v[r, :]

            def col(c, _):
                sl = pl.ds(c * 16, 16)
                a_v[r, sl] = s0 * a_v[r, sl] + s1 * b_v[r, sl]
                return 0

            lax.fori_loop(0, H // 16, col, 0)
            return 0

        lax.fori_loop(0, _CHUNK, row, 0)
        pltpu.sync_copy(a_v, out_hbm.at[pl.ds(base, _CHUNK)])


def _sc_combine(y, d0, d1, w0, w1):
    mesh = plsc.VectorSubcoreMesh(core_axis_name="c", subcore_axis_name="s")
    return pl.kernel(
        _sc_combine_body,
        out_type=jax.ShapeDtypeStruct((S, H), jnp.float32),
        mesh=mesh,
        scratch_types=[
            pltpu.VMEM((_CHUNK,), jnp.int32),
            pltpu.VMEM((_CHUNK,), jnp.int32),
            pltpu.VMEM((_CHUNK, 16), jnp.float32),
            pltpu.VMEM((_CHUNK, 16), jnp.float32),
            pltpu.VMEM((_CHUNK, H), jnp.float32),
            pltpu.VMEM((_CHUNK, H), jnp.float32),
            pltpu.SemaphoreType.DMA,
        ],
    )(y, d0, d1, w0, w1)


def kernel(hidden_states, W_router, Wg, Wu, Wd):
    x = hidden_states.reshape(S, H)
    d0c, d1c, w0c, w1c, metac, auxc = _router(x, W_router)
    d0 = d0c.reshape(S)
    d1 = d1c.reshape(S)
    meta = metac.reshape(2 * E)
    y = _sc_dispatch(x, d0, d1)
    z = _ffn(meta, y, Wg, Wu, Wd)
    out = _sc_combine(z, d0, d1, w0c, w1c)
    return out.reshape(1, S, H), auxc.reshape(())


# speedup vs baseline: 2.2755x; 1.5576x over previous
import functools

import jax
import jax.numpy as jnp
from jax import lax
from jax.experimental import pallas as pl
from jax.experimental.pallas import tpu as pltpu
from jax.experimental.pallas import tpu_sc as plsc

E = 8
S = 2048
H = 1024
I = 4096
TB = 256
NBLK = 23
RC = 128
PADTOT = NBLK * TB
IT = 4
TI = I // IT

_NEG_INF = float("-inf")


def _router_body(x_ref, wr_ref, d0_ref, d1_ref, w0_ref, w1_ref, meta_ref,
                 aux_ref, oh_ref):
    x = x_ref[...]
    logits = lax.dot_general(x, wr_ref[...], (((1,), (1,)), ((), ())),
                             preferred_element_type=jnp.float32)
    iota8 = lax.broadcasted_iota(jnp.int32, (S, E), 1).astype(jnp.float32)
    v0 = jnp.max(logits, axis=1, keepdims=True)
    i0 = jnp.min(jnp.where(logits == v0, iota8, 8.0), axis=1, keepdims=True)
    oh0 = (iota8 == i0).astype(jnp.float32)
    masked = jnp.where(iota8 == i0, _NEG_INF, logits)
    v1 = jnp.max(masked, axis=1, keepdims=True)
    i1 = jnp.min(jnp.where(masked == v1, iota8, 8.0), axis=1, keepdims=True)
    oh1 = (iota8 == i1).astype(jnp.float32)

    g = jnp.exp(v1 - v0)
    s = 1.0 / (1.0 + g)
    w0_ref[...] = jnp.broadcast_to(s, (S, 16))
    w1_ref[...] = jnp.broadcast_to(g * s, (S, 16))

    p = jnp.exp(logits - v0)
    probs = p / jnp.sum(p, axis=1, keepdims=True)
    probs_mean = jnp.sum(probs, axis=0, keepdims=True) * (1.0 / S)
    counts = jnp.sum(oh0 + oh1, axis=0, keepdims=True)
    usage = counts * (1.0 / (S * 2))
    aux_ref[...] = jnp.sum(probs_mean * usage, axis=1, keepdims=True) * float(E)

    nblk = jnp.floor((counts + (TB - 1)) * (1.0 / TB))
    tri8 = (lax.broadcasted_iota(jnp.int32, (E, E), 0)
            < lax.broadcasted_iota(jnp.int32, (E, E), 1)).astype(jnp.float32)
    blk_excl = lax.dot_general(nblk, tri8, (((1,), (0,)), ((), ())),
                               preferred_element_type=jnp.float32)
    pad_off = blk_excl * float(TB)
    nblocks = jnp.sum(nblk, axis=1, keepdims=True)

    meta_ref[:, pl.ds(0, E)] = nblk.astype(jnp.int32)
    meta_ref[:, pl.ds(E, E)] = blk_excl.astype(jnp.int32)

    oh_ref[:, pl.ds(0, E)] = oh0
    oh_ref[:, pl.ds(E, E)] = oh1
    c0tot = jnp.sum(oh0, axis=0, keepdims=True)
    ltri = (lax.broadcasted_iota(jnp.int32, (RC, RC), 1)
            < lax.broadcasted_iota(jnp.int32, (RC, RC), 0)).astype(jnp.bfloat16)

    def chunk(ch, carry):
        ohc = oh_ref[pl.ds(ch * RC, RC), :]
        csc = lax.dot_general(ltri, ohc.astype(jnp.bfloat16),
                              (((1,), (0,)), ((), ())),
                              preferred_element_type=jnp.float32) + carry
        rank0 = csc[:, 0:E]
        rank1 = csc[:, E:2 * E] + c0tot
        d0 = jnp.sum(ohc[:, 0:E] * (pad_off + rank0), axis=1, keepdims=True)
        d1 = jnp.sum(ohc[:, E:2 * E] * (pad_off + rank1), axis=1, keepdims=True)
        d0_ref[pl.ds(ch * RC, RC), :] = d0.astype(jnp.int32)
        d1_ref[pl.ds(ch * RC, RC), :] = d1.astype(jnp.int32)
        return carry + jnp.sum(ohc, axis=0, keepdims=True)

    lax.fori_loop(0, S // RC, chunk, jnp.zeros((1, 2 * E), jnp.float32))


def _router(x, w_router):
    return pl.pallas_call(
        _router_body,
        out_shape=(
            jax.ShapeDtypeStruct((S, 1), jnp.int32),
            jax.ShapeDtypeStruct((S, 1), jnp.int32),
            jax.ShapeDtypeStruct((S, 16), jnp.float32),
            jax.ShapeDtypeStruct((S, 16), jnp.float32),
            jax.ShapeDtypeStruct((1, 2 * E), jnp.int32),
            jax.ShapeDtypeStruct((1, 1), jnp.float32),
        ),
        scratch_shapes=[pltpu.VMEM((S, 2 * E), jnp.float32)],
    )(x, w_router)


_TOK_PER_W = S // 32


def _sc_dispatch_body(x_hbm, d0_hbm, d1_hbm, y_hbm, i0_v, i1_v, x_v, sem):
    wid = lax.axis_index("s") * 2 + lax.axis_index("c")
    base = wid * _TOK_PER_W
    pltpu.sync_copy(d0_hbm.at[pl.ds(base, _TOK_PER_W)], i0_v)
    pltpu.sync_copy(d1_hbm.at[pl.ds(base, _TOK_PER_W)], i1_v)
    pltpu.sync_copy(x_hbm.at[pl.ds(base, _TOK_PER_W)], x_v)
    pltpu.async_copy(x_v, y_hbm.at[i0_v], sem).wait()
    pltpu.async_copy(x_v, y_hbm.at[i1_v], sem).wait()


def _sc_dispatch(x, d0, d1):
    mesh = plsc.VectorSubcoreMesh(core_axis_name="c", subcore_axis_name="s")
    return pl.kernel(
        _sc_dispatch_body,
        out_type=jax.ShapeDtypeStruct((PADTOT, H), jnp.float32),
        mesh=mesh,
        scratch_types=[
            pltpu.VMEM((_TOK_PER_W,), jnp.int32),
            pltpu.VMEM((_TOK_PER_W,), jnp.int32),
            pltpu.VMEM((_TOK_PER_W, H), jnp.float32),
            pltpu.SemaphoreType.DMA,
        ],
    )(x, d0, d1)


_MAX_ROWS = 2048


def _ffn_body(meta_ref, xg_hbm, wg_ref, wu_ref, wd_ref, z_hbm,
              xbf_ref, acc_ref, inbuf_ref, outbuf_ref, insem, outsem):
    e = pl.program_id(0)
    it = pl.program_id(1)
    nb = meta_ref[e]
    roff = meta_ref[E + e] * TB

    wgb = wg_ref[0].astype(jnp.bfloat16)
    wub = wu_ref[0].astype(jnp.bfloat16)
    wdb = wd_ref[0].astype(jnp.bfloat16)

    def start_in(j, slot):
        pltpu.make_async_copy(
            xg_hbm.at[pl.ds(roff + j * TB, TB), :],
            inbuf_ref.at[slot], insem.at[slot]).start()

    def wait_in(slot):
        pltpu.make_async_copy(
            xg_hbm.at[pl.ds(0, TB), :],
            inbuf_ref.at[slot], insem.at[slot]).wait()

    def wait_out(slot):
        pltpu.make_async_copy(
            outbuf_ref.at[slot],
            z_hbm.at[pl.ds(0, TB), :], outsem.at[slot]).wait()

    def compute(j, xc):
        g = lax.dot_general(xc, wgb, (((1,), (1,)), ((), ())),
                            preferred_element_type=jnp.float32)
        u = lax.dot_general(xc, wub, (((1,), (1,)), ((), ())),
                            preferred_element_type=jnp.float32)
        h = (g * jax.nn.sigmoid(g)) * u
        part = lax.dot_general(h.astype(jnp.bfloat16), wdb,
                               (((1,), (1,)), ((), ())),
                               preferred_element_type=jnp.float32)
        sl = pl.ds(j * TB, TB)

        @pl.when(it == 0)
        def _():
            acc_ref[sl, :] = part

        @pl.when(jnp.logical_and(it > 0, it < IT - 1))
        def _():
            acc_ref[sl, :] += part

        @pl.when(it == IT - 1)
        def _():
            res = acc_ref[sl, :] + part
            slot = lax.rem(j, 2)

            @pl.when(j >= 2)
            def _():
                wait_out(slot)

            outbuf_ref[slot] = res
            pltpu.make_async_copy(
                outbuf_ref.at[slot],
                z_hbm.at[pl.ds(roff + j * TB, TB), :],
                outsem.at[slot]).start()

    @pl.when(it == 0)
    def _():
        @pl.when(nb > 0)
        def _():
            start_in(0, 0)

        @pl.when(nb > 1)
        def _():
            start_in(1, 1)

        def body0(j, c):
            slot = lax.rem(j, 2)
            wait_in(slot)
            xc = inbuf_ref[slot].astype(jnp.bfloat16)
            xbf_ref[pl.ds(j * TB, TB), :] = xc

            @pl.when(j + 2 < nb)
            def _():
                start_in(j + 2, slot)

            compute(j, xc)
            return c

        lax.fori_loop(0, nb, body0, 0)

    @pl.when(it != 0)
    def _():
        def bodyn(j, c):
            compute(j, xbf_ref[pl.ds(j * TB, TB), :])
            return c

        lax.fori_loop(0, nb, bodyn, 0)

    @pl.when(it == IT - 1)
    def _():
        @pl.when(nb >= 2)
        def _():
            wait_out(lax.rem(nb, 2))

        @pl.when(nb >= 1)
        def _():
            wait_out(lax.rem(nb - 1, 2))


def _ffn(meta, y, wg, wu, wd):
    grid_spec = pltpu.PrefetchScalarGridSpec(
        num_scalar_prefetch=1,
        grid=(E, IT),
        in_specs=[
            pl.BlockSpec(memory_space=pltpu.HBM),
            pl.BlockSpec((1, TI, H), lambda e, it, m: (e, it, 0)),
            pl.BlockSpec((1, TI, H), lambda e, it, m: (e, it, 0)),
            pl.BlockSpec((1, H, TI), lambda e, it, m: (e, 0, it)),
        ],
        out_specs=pl.BlockSpec(memory_space=pltpu.HBM),
        scratch_shapes=[
            pltpu.VMEM((_MAX_ROWS, H), jnp.bfloat16),
            pltpu.VMEM((_MAX_ROWS, H), jnp.float32),
            pltpu.VMEM((2, TB, H), jnp.float32),
            pltpu.VMEM((2, TB, H), jnp.float32),
            pltpu.SemaphoreType.DMA((2,)),
            pltpu.SemaphoreType.DMA((2,)),
        ],
    )
    return pl.pallas_call(
        _ffn_body,
        grid_spec=grid_spec,
        out_shape=jax.ShapeDtypeStruct((PADTOT, H), jnp.float32),
    )(meta, y, wg, wu, wd)


_CHUNK = 32


def _sc_combine_body(y_hbm, d0_hbm, d1_hbm, w0_hbm, w1_hbm, out_hbm,
                     i0_v, i1_v, w0_v, w1_v, a_v, b_v, sem):
    wid = lax.axis_index("s") * 2 + lax.axis_index("c")
    for half in range(_TOK_PER_W // _CHUNK):
        base = wid * _TOK_PER_W + half * _CHUNK
        pltpu.sync_copy(d0_hbm.at[pl.ds(base, _CHUNK)], i0_v)
        pltpu.sync_copy(d1_hbm.at[pl.ds(base, _CHUNK)], i1_v)
        pltpu.sync_copy(w0_hbm.at[pl.ds(base, _CHUNK)], w0_v)
        pltpu.sync_copy(w1_hbm.at[pl.ds(base, _CHUNK)], w1_v)
        pltpu.async_copy(y_hbm.at[i0_v], a_v, sem).wait()
        pltpu.async_copy(y_hbm.at[i1_v], b_v, sem).wait()

        def row(r, _):
            s0 = w0_v[r, :]
            s1 = w1_v[r, :]

            def col(c, _):
                sl = pl.ds(c * 16, 16)
                a_v[r, sl] = s0 * a_v[r, sl] + s1 * b_v[r, sl]
                return 0

            lax.fori_loop(0, H // 16, col, 0)
            return 0

        lax.fori_loop(0, _CHUNK, row, 0)
        pltpu.sync_copy(a_v, out_hbm.at[pl.ds(base, _CHUNK)])


def _sc_combine(y, d0, d1, w0, w1):
    mesh = plsc.VectorSubcoreMesh(core_axis_name="c", subcore_axis_name="s")
    return pl.kernel(
        _sc_combine_body,
        out_type=jax.ShapeDtypeStruct((S, H), jnp.float32),
        mesh=mesh,
        scratch_types=[
            pltpu.VMEM((_CHUNK,), jnp.int32),
            pltpu.VMEM((_CHUNK,), jnp.int32),
            pltpu.VMEM((_CHUNK, 16), jnp.float32),
            pltpu.VMEM((_CHUNK, 16), jnp.float32),
            pltpu.VMEM((_CHUNK, H), jnp.float32),
            pltpu.VMEM((_CHUNK, H), jnp.float32),
            pltpu.SemaphoreType.DMA,
        ],
    )(y, d0, d1, w0, w1)


def kernel(hidden_states, W_router, Wg, Wu, Wd):
    x = hidden_states.reshape(S, H)
    d0c, d1c, w0c, w1c, metac, auxc = _router(x, W_router)
    d0 = d0c.reshape(S)
    d1 = d1c.reshape(S)
    meta = metac.reshape(2 * E)
    y = _sc_dispatch(x, d0, d1)
    z = _ffn(meta, y, Wg, Wu, Wd)
    out = _sc_combine(z, d0, d1, w0c, w1c)
    return out.reshape(1, S, H), auxc.reshape(())


# speedup vs baseline: 2.4120x; 1.0600x over previous
import functools

import jax
import jax.numpy as jnp
from jax import lax
from jax.experimental import pallas as pl
from jax.experimental.pallas import tpu as pltpu
from jax.experimental.pallas import tpu_sc as plsc

E = 8
S = 2048
H = 1024
I = 4096
TB = 256
NBLK = 23
RC = 128
PADTOT = NBLK * TB
IT = 4
TI = I // IT

_NEG_INF = float("-inf")


def _router_body(x_ref, wr_ref, d0_ref, d1_ref, w0_ref, w1_ref, meta_ref,
                 aux_ref, oh_ref):
    x = x_ref[...]
    logits = lax.dot_general(x, wr_ref[...], (((1,), (1,)), ((), ())),
                             preferred_element_type=jnp.float32)
    iota8 = lax.broadcasted_iota(jnp.int32, (S, E), 1).astype(jnp.float32)
    v0 = jnp.max(logits, axis=1, keepdims=True)
    i0 = jnp.min(jnp.where(logits == v0, iota8, 8.0), axis=1, keepdims=True)
    oh0 = (iota8 == i0).astype(jnp.float32)
    masked = jnp.where(iota8 == i0, _NEG_INF, logits)
    v1 = jnp.max(masked, axis=1, keepdims=True)
    i1 = jnp.min(jnp.where(masked == v1, iota8, 8.0), axis=1, keepdims=True)
    oh1 = (iota8 == i1).astype(jnp.float32)

    g = jnp.exp(v1 - v0)
    s = 1.0 / (1.0 + g)
    w0_ref[...] = jnp.broadcast_to(s, (S, 16))
    w1_ref[...] = jnp.broadcast_to(g * s, (S, 16))

    p = jnp.exp(logits - v0)
    probs = p / jnp.sum(p, axis=1, keepdims=True)
    probs_mean = jnp.sum(probs, axis=0, keepdims=True) * (1.0 / S)
    counts = jnp.sum(oh0 + oh1, axis=0, keepdims=True)
    usage = counts * (1.0 / (S * 2))
    aux_ref[...] = jnp.sum(probs_mean * usage, axis=1, keepdims=True) * float(E)

    nblk = jnp.floor((counts + (TB - 1)) * (1.0 / TB))
    tri8 = (lax.broadcasted_iota(jnp.int32, (E, E), 0)
            < lax.broadcasted_iota(jnp.int32, (E, E), 1)).astype(jnp.float32)
    blk_excl = lax.dot_general(nblk, tri8, (((1,), (0,)), ((), ())),
                               preferred_element_type=jnp.float32)
    pad_off = blk_excl * float(TB)
    nblocks = jnp.sum(nblk, axis=1, keepdims=True)

    meta_ref[:, pl.ds(0, E)] = nblk.astype(jnp.int32)
    meta_ref[:, pl.ds(E, E)] = blk_excl.astype(jnp.int32)

    oh_ref[:, pl.ds(0, E)] = oh0
    oh_ref[:, pl.ds(E, E)] = oh1
    c0tot = jnp.sum(oh0, axis=0, keepdims=True)
    ltri = (lax.broadcasted_iota(jnp.int32, (RC, RC), 1)
            < lax.broadcasted_iota(jnp.int32, (RC, RC), 0)).astype(jnp.bfloat16)

    def chunk(ch, carry):
        ohc = oh_ref[pl.ds(ch * RC, RC), :]
        csc = lax.dot_general(ltri, ohc.astype(jnp.bfloat16),
                              (((1,), (0,)), ((), ())),
                              preferred_element_type=jnp.float32) + carry
        rank0 = csc[:, 0:E]
        rank1 = csc[:, E:2 * E] + c0tot
        d0 = jnp.sum(ohc[:, 0:E] * (pad_off + rank0), axis=1, keepdims=True)
        d1 = jnp.sum(ohc[:, E:2 * E] * (pad_off + rank1), axis=1, keepdims=True)
        d0_ref[pl.ds(ch * RC, RC), :] = d0.astype(jnp.int32)
        d1_ref[pl.ds(ch * RC, RC), :] = d1.astype(jnp.int32)
        return carry + jnp.sum(ohc, axis=0, keepdims=True)

    lax.fori_loop(0, S // RC, chunk, jnp.zeros((1, 2 * E), jnp.float32))


def _router(x, w_router):
    return pl.pallas_call(
        _router_body,
        out_shape=(
            jax.ShapeDtypeStruct((S, 1), jnp.int32),
            jax.ShapeDtypeStruct((S, 1), jnp.int32),
            jax.ShapeDtypeStruct((S, 16), jnp.float32),
            jax.ShapeDtypeStruct((S, 16), jnp.float32),
            jax.ShapeDtypeStruct((1, 2 * E), jnp.int32),
            jax.ShapeDtypeStruct((1, 1), jnp.float32),
        ),
        scratch_shapes=[pltpu.VMEM((S, 2 * E), jnp.float32)],
    )(x, w_router)


_TOK_PER_W = S // 32


def _sc_dispatch_body(x_hbm, d0_hbm, d1_hbm, y_hbm, i0_v, i1_v, x_v, sem):
    wid = lax.axis_index("s") * 2 + lax.axis_index("c")
    base = wid * _TOK_PER_W
    pltpu.sync_copy(d0_hbm.at[pl.ds(base, _TOK_PER_W)], i0_v)
    pltpu.sync_copy(d1_hbm.at[pl.ds(base, _TOK_PER_W)], i1_v)
    pltpu.sync_copy(x_hbm.at[pl.ds(base, _TOK_PER_W)], x_v)
    pltpu.async_copy(x_v, y_hbm.at[i0_v], sem).wait()
    pltpu.async_copy(x_v, y_hbm.at[i1_v], sem).wait()


def _sc_dispatch(x, d0, d1):
    mesh = plsc.VectorSubcoreMesh(core_axis_name="c", subcore_axis_name="s")
    return pl.kernel(
        _sc_dispatch_body,
        out_type=jax.ShapeDtypeStruct((PADTOT, H), jnp.float32),
        mesh=mesh,
        scratch_types=[
            pltpu.VMEM((_TOK_PER_W,), jnp.int32),
            pltpu.VMEM((_TOK_PER_W,), jnp.int32),
            pltpu.VMEM((_TOK_PER_W, H), jnp.float32),
            pltpu.SemaphoreType.DMA,
        ],
    )(x, d0, d1)


_MAX_ROWS = 2048


def _ffn_body(meta_ref, xg_hbm, wg_ref, wu_ref, wd_ref, z_hbm,
              xbf_ref, acc_ref, inbuf_ref, outbuf_ref, insem, outsem):
    e = pl.program_id(0)
    it = pl.program_id(1)
    nb = meta_ref[e]
    roff = meta_ref[E + e] * TB

    wgb = wg_ref[0].astype(jnp.bfloat16)
    wub = wu_ref[0].astype(jnp.bfloat16)
    wdb = wd_ref[0].astype(jnp.bfloat16)

    def start_in(j, slot):
        pltpu.make_async_copy(
            xg_hbm.at[pl.ds(roff + j * TB, TB), :],
            inbuf_ref.at[slot], insem.at[slot]).start()

    def wait_in(slot):
        pltpu.make_async_copy(
            xg_hbm.at[pl.ds(0, TB), :],
            inbuf_ref.at[slot], insem.at[slot]).wait()

    def wait_out(slot):
        pltpu.make_async_copy(
            outbuf_ref.at[slot],
            z_hbm.at[pl.ds(0, TB), :], outsem.at[slot]).wait()

    def compute(j, xc):
        g = lax.dot_general(xc, wgb, (((1,), (1,)), ((), ())),
                            preferred_element_type=jnp.float32)
        u = lax.dot_general(xc, wub, (((1,), (1,)), ((), ())),
                            preferred_element_type=jnp.float32)
        h = (g * jax.nn.sigmoid(g)) * u
        part = lax.dot_general(h.astype(jnp.bfloat16), wdb,
                               (((1,), (1,)), ((), ())),
                               preferred_element_type=jnp.float32)
        sl = pl.ds(j * TB, TB)

        @pl.when(it == 0)
        def _():
            acc_ref[sl, :] = part

        @pl.when(jnp.logical_and(it > 0, it < IT - 1))
        def _():
            acc_ref[sl, :] += part

        @pl.when(it == IT - 1)
        def _():
            res = acc_ref[sl, :] + part
            slot = lax.rem(j, 2)

            @pl.when(j >= 2)
            def _():
                wait_out(slot)

            outbuf_ref[slot] = res
            pltpu.make_async_copy(
                outbuf_ref.at[slot],
                z_hbm.at[pl.ds(roff + j * TB, TB), :],
                outsem.at[slot]).start()

    @pl.when(it == 0)
    def _():
        @pl.when(nb > 0)
        def _():
            start_in(0, 0)

        @pl.when(nb > 1)
        def _():
            start_in(1, 1)

        def body0(j, c):
            slot = lax.rem(j, 2)
            wait_in(slot)
            xc = inbuf_ref[slot].astype(jnp.bfloat16)
            xbf_ref[pl.ds(j * TB, TB), :] = xc

            @pl.when(j + 2 < nb)
            def _():
                start_in(j + 2, slot)

            compute(j, xc)
            return c

        lax.fori_loop(0, nb, body0, 0)

    @pl.when(it != 0)
    def _():
        def bodyn(j, c):
            compute(j, xbf_ref[pl.ds(j * TB, TB), :])
            return c

        lax.fori_loop(0, nb, bodyn, 0)

    @pl.when(it == IT - 1)
    def _():
        @pl.when(nb >= 2)
        def _():
            wait_out(lax.rem(nb, 2))

        @pl.when(nb >= 1)
        def _():
            wait_out(lax.rem(nb - 1, 2))


def _ffn(meta, y, wg, wu, wd):
    grid_spec = pltpu.PrefetchScalarGridSpec(
        num_scalar_prefetch=1,
        grid=(E, IT),
        in_specs=[
            pl.BlockSpec(memory_space=pltpu.HBM),
            pl.BlockSpec((1, TI, H), lambda e, it, m: (e, it, 0)),
            pl.BlockSpec((1, TI, H), lambda e, it, m: (e, it, 0)),
            pl.BlockSpec((1, H, TI), lambda e, it, m: (e, 0, it)),
        ],
        out_specs=pl.BlockSpec(memory_space=pltpu.HBM),
        scratch_shapes=[
            pltpu.VMEM((_MAX_ROWS, H), jnp.bfloat16),
            pltpu.VMEM((_MAX_ROWS, H), jnp.float32),
            pltpu.VMEM((2, TB, H), jnp.float32),
            pltpu.VMEM((2, TB, H), jnp.float32),
            pltpu.SemaphoreType.DMA((2,)),
            pltpu.SemaphoreType.DMA((2,)),
        ],
    )
    return pl.pallas_call(
        _ffn_body,
        grid_spec=grid_spec,
        out_shape=jax.ShapeDtypeStruct((PADTOT, H), jnp.float32),
    )(meta, y, wg, wu, wd)


_CH = 16
_NCH = _TOK_PER_W // _CH


def _sc_combine_body(y_hbm, d0_hbm, d1_hbm, w0_hbm, w1_hbm, out_hbm,
                     i0_v, i1_v, w0_v, w1_v, a_v, b_v, o_v, gsa, gsb, osem):
    wid = lax.axis_index("s") * 2 + lax.axis_index("c")
    base = wid * _TOK_PER_W
    pltpu.sync_copy(d0_hbm.at[pl.ds(base, _TOK_PER_W)], i0_v)
    pltpu.sync_copy(d1_hbm.at[pl.ds(base, _TOK_PER_W)], i1_v)
    pltpu.sync_copy(w0_hbm.at[pl.ds(base, _TOK_PER_W)], w0_v)
    pltpu.sync_copy(w1_hbm.at[pl.ds(base, _TOK_PER_W)], w1_v)

    def gather(c, slot):
        return (
            pltpu.make_async_copy(y_hbm.at[i0_v.at[pl.ds(c * _CH, _CH)]],
                                  a_v.at[slot], gsa.at[slot]),
            pltpu.make_async_copy(y_hbm.at[i1_v.at[pl.ds(c * _CH, _CH)]],
                                  b_v.at[slot], gsb.at[slot]),
        )

    def writeback(c, slot):
        return pltpu.make_async_copy(
            o_v.at[slot], out_hbm.at[pl.ds(base + c * _CH, _CH)],
            osem.at[slot])

    ga, gb = gather(0, 0)
    ga.start()
    gb.start()
    for c in range(_NCH):
        slot = c % 2
        ga, gb = gather(c, slot)
        ga.wait()
        gb.wait()
        if c + 1 < _NCH:
            ga1, gb1 = gather(c + 1, 1 - slot)
            ga1.start()
            gb1.start()
        if c >= 2:
            writeback(c - 2, slot).wait()

        def row(r, _):
            s0 = w0_v[c * _CH + r, :]
            s1 = w1_v[c * _CH + r, :]

            def col(k, _):
                sl = pl.ds(k * 16, 16)
                o_v[slot, r, sl] = s0 * a_v[slot, r, sl] + s1 * b_v[slot, r, sl]
                return 0

            lax.fori_loop(0, H // 16, col, 0)
            return 0

        lax.fori_loop(0, _CH, row, 0)
        writeback(c, slot).start()
    writeback(_NCH - 2, 0).wait()
    writeback(_NCH - 1, 1).wait()


def _sc_combine(y, d0, d1, w0, w1):
    mesh = plsc.VectorSubcoreMesh(core_axis_name="c", subcore_axis_name="s")
    return pl.kernel(
        _sc_combine_body,
        out_type=jax.ShapeDtypeStruct((S, H), jnp.float32),
        mesh=mesh,
        scratch_types=[
            pltpu.VMEM((_TOK_PER_W,), jnp.int32),
            pltpu.VMEM((_TOK_PER_W,), jnp.int32),
            pltpu.VMEM((_TOK_PER_W, 16), jnp.float32),
            pltpu.VMEM((_TOK_PER_W, 16), jnp.float32),
            pltpu.VMEM((2, _CH, H), jnp.float32),
            pltpu.VMEM((2, _CH, H), jnp.float32),
            pltpu.VMEM((2, _CH, H), jnp.float32),
            pltpu.SemaphoreType.DMA((2,)),
            pltpu.SemaphoreType.DMA((2,)),
            pltpu.SemaphoreType.DMA((2,)),
        ],
    )(y, d0, d1, w0, w1)


def kernel(hidden_states, W_router, Wg, Wu, Wd):
    x = hidden_states.reshape(S, H)
    d0c, d1c, w0c, w1c, metac, auxc = _router(x, W_router)
    d0 = d0c.reshape(S)
    d1 = d1c.reshape(S)
    meta = metac.reshape(2 * E)
    y = _sc_dispatch(x, d0, d1)
    z = _ffn(meta, y, Wg, Wu, Wd)
    out = _sc_combine(z, d0, d1, w0c, w1c)
    return out.reshape(1, S, H), auxc.reshape(())


# speedup vs baseline: 2.4391x; 1.0112x over previous
import functools

import jax
import jax.numpy as jnp
from jax import lax
from jax.experimental import pallas as pl
from jax.experimental.pallas import tpu as pltpu
from jax.experimental.pallas import tpu_sc as plsc

E = 8
S = 2048
H = 1024
I = 4096
TB = 256
NBLK = 23
RC = 128
PADTOT = NBLK * TB
IT = 4
TI = I // IT

_NEG_INF = float("-inf")


def _router_body(x_ref, wr_ref, d0_ref, d1_ref, w0_ref, w1_ref, meta_ref,
                 aux_ref, oh_ref):
    x = x_ref[...]
    logits = lax.dot_general(x, wr_ref[...], (((1,), (1,)), ((), ())),
                             preferred_element_type=jnp.float32)
    iota8 = lax.broadcasted_iota(jnp.int32, (S, E), 1).astype(jnp.float32)
    v0 = jnp.max(logits, axis=1, keepdims=True)
    i0 = jnp.min(jnp.where(logits == v0, iota8, 8.0), axis=1, keepdims=True)
    oh0 = (iota8 == i0).astype(jnp.float32)
    masked = jnp.where(iota8 == i0, _NEG_INF, logits)
    v1 = jnp.max(masked, axis=1, keepdims=True)
    i1 = jnp.min(jnp.where(masked == v1, iota8, 8.0), axis=1, keepdims=True)
    oh1 = (iota8 == i1).astype(jnp.float32)

    g = jnp.exp(v1 - v0)
    s = 1.0 / (1.0 + g)
    w0_ref[...] = jnp.broadcast_to(s, (S, 16))
    w1_ref[...] = jnp.broadcast_to(g * s, (S, 16))

    p = jnp.exp(logits - v0)
    probs = p / jnp.sum(p, axis=1, keepdims=True)
    probs_mean = jnp.sum(probs, axis=0, keepdims=True) * (1.0 / S)
    counts = jnp.sum(oh0 + oh1, axis=0, keepdims=True)
    usage = counts * (1.0 / (S * 2))
    aux_ref[...] = jnp.sum(probs_mean * usage, axis=1, keepdims=True) * float(E)

    nblk = jnp.floor((counts + (TB - 1)) * (1.0 / TB))
    tri8 = (lax.broadcasted_iota(jnp.int32, (E, E), 0)
            < lax.broadcasted_iota(jnp.int32, (E, E), 1)).astype(jnp.float32)
    blk_excl = lax.dot_general(nblk, tri8, (((1,), (0,)), ((), ())),
                               preferred_element_type=jnp.float32)
    pad_off = blk_excl * float(TB)
    nblocks = jnp.sum(nblk, axis=1, keepdims=True)

    meta_ref[:, pl.ds(0, E)] = nblk.astype(jnp.int32)
    meta_ref[:, pl.ds(E, E)] = blk_excl.astype(jnp.int32)

    oh_ref[:, pl.ds(0, E)] = oh0
    oh_ref[:, pl.ds(E, E)] = oh1
    c0tot = jnp.sum(oh0, axis=0, keepdims=True)
    ltri = (lax.broadcasted_iota(jnp.int32, (RC, RC), 1)
            < lax.broadcasted_iota(jnp.int32, (RC, RC), 0)).astype(jnp.bfloat16)

    def chunk(ch, carry):
        ohc = oh_ref[pl.ds(ch * RC, RC), :]
        csc = lax.dot_general(ltri, ohc.astype(jnp.bfloat16),
                              (((1,), (0,)), ((), ())),
                              preferred_element_type=jnp.float32) + carry
        rank0 = csc[:, 0:E]
        rank1 = csc[:, E:2 * E] + c0tot
        d0 = jnp.sum(ohc[:, 0:E] * (pad_off + rank0), axis=1, keepdims=True)
        d1 = jnp.sum(ohc[:, E:2 * E] * (pad_off + rank1), axis=1, keepdims=True)
        d0_ref[pl.ds(ch * RC, RC)] = d0.astype(jnp.int32).reshape(RC)
        d1_ref[pl.ds(ch * RC, RC)] = d1.astype(jnp.int32).reshape(RC)
        return carry + jnp.sum(ohc, axis=0, keepdims=True)

    lax.fori_loop(0, S // RC, chunk, jnp.zeros((1, 2 * E), jnp.float32))


def _router(x, w_router):
    return pl.pallas_call(
        _router_body,
        out_shape=(
            jax.ShapeDtypeStruct((S,), jnp.int32),
            jax.ShapeDtypeStruct((S,), jnp.int32),
            jax.ShapeDtypeStruct((S, 16), jnp.float32),
            jax.ShapeDtypeStruct((S, 16), jnp.float32),
            jax.ShapeDtypeStruct((1, 2 * E), jnp.int32),
            jax.ShapeDtypeStruct((1, 1), jnp.float32),
        ),
        scratch_shapes=[pltpu.VMEM((S, 2 * E), jnp.float32)],
    )(x, w_router)


_TOK_PER_W = S // 32


def _sc_dispatch_body(x_hbm, d0_hbm, d1_hbm, y_hbm, i0_v, i1_v, x_v, sem):
    wid = lax.axis_index("s") * 2 + lax.axis_index("c")
    base = wid * _TOK_PER_W
    pltpu.sync_copy(d0_hbm.at[pl.ds(base, _TOK_PER_W)], i0_v)
    pltpu.sync_copy(d1_hbm.at[pl.ds(base, _TOK_PER_W)], i1_v)
    pltpu.sync_copy(x_hbm.at[pl.ds(base, _TOK_PER_W)], x_v)
    pltpu.async_copy(x_v, y_hbm.at[i0_v], sem).wait()
    pltpu.async_copy(x_v, y_hbm.at[i1_v], sem).wait()


def _sc_dispatch(x, d0, d1):
    mesh = plsc.VectorSubcoreMesh(core_axis_name="c", subcore_axis_name="s")
    return pl.kernel(
        _sc_dispatch_body,
        out_type=jax.ShapeDtypeStruct((PADTOT, H), jnp.float32),
        mesh=mesh,
        scratch_types=[
            pltpu.VMEM((_TOK_PER_W,), jnp.int32),
            pltpu.VMEM((_TOK_PER_W,), jnp.int32),
            pltpu.VMEM((_TOK_PER_W, H), jnp.float32),
            pltpu.SemaphoreType.DMA,
        ],
    )(x, d0, d1)


_MAX_ROWS = 2048


def _ffn_body(meta_ref, xg_hbm, wg_ref, wu_ref, wd_ref, z_hbm,
              xbf_ref, acc_ref, inbuf_ref, outbuf_ref, insem, outsem):
    e = pl.program_id(0)
    it = pl.program_id(1)
    nb = meta_ref[e]
    roff = meta_ref[E + e] * TB

    wgb = wg_ref[0].astype(jnp.bfloat16)
    wub = wu_ref[0].astype(jnp.bfloat16)
    wdb = wd_ref[0].astype(jnp.bfloat16)

    def start_in(j, slot):
        pltpu.make_async_copy(
            xg_hbm.at[pl.ds(roff + j * TB, TB), :],
            inbuf_ref.at[slot], insem.at[slot]).start()

    def wait_in(slot):
        pltpu.make_async_copy(
            xg_hbm.at[pl.ds(0, TB), :],
            inbuf_ref.at[slot], insem.at[slot]).wait()

    def wait_out(slot):
        pltpu.make_async_copy(
            outbuf_ref.at[slot],
            z_hbm.at[pl.ds(0, TB), :], outsem.at[slot]).wait()

    def compute(j, xc):
        g = lax.dot_general(xc, wgb, (((1,), (1,)), ((), ())),
                            preferred_element_type=jnp.float32)
        u = lax.dot_general(xc, wub, (((1,), (1,)), ((), ())),
                            preferred_element_type=jnp.float32)
        h = (g * jax.nn.sigmoid(g)) * u
        part = lax.dot_general(h.astype(jnp.bfloat16), wdb,
                               (((1,), (1,)), ((), ())),
                               preferred_element_type=jnp.float32)
        sl = pl.ds(j * TB, TB)

        @pl.when(it == 0)
        def _():
            acc_ref[sl, :] = part

        @pl.when(jnp.logical_and(it > 0, it < IT - 1))
        def _():
            acc_ref[sl, :] += part

        @pl.when(it == IT - 1)
        def _():
            res = acc_ref[sl, :] + part
            slot = lax.rem(j, 2)

            @pl.when(j >= 2)
            def _():
                wait_out(slot)

            outbuf_ref[slot] = res
            pltpu.make_async_copy(
                outbuf_ref.at[slot],
                z_hbm.at[pl.ds(roff + j * TB, TB), :],
                outsem.at[slot]).start()

    @pl.when(it == 0)
    def _():
        @pl.when(nb > 0)
        def _():
            start_in(0, 0)

        @pl.when(nb > 1)
        def _():
            start_in(1, 1)

        def body0(j, c):
            slot = lax.rem(j, 2)
            wait_in(slot)
            xc = inbuf_ref[slot].astype(jnp.bfloat16)
            xbf_ref[pl.ds(j * TB, TB), :] = xc

            @pl.when(j + 2 < nb)
            def _():
                start_in(j + 2, slot)

            compute(j, xc)
            return c

        lax.fori_loop(0, nb, body0, 0)

    @pl.when(it != 0)
    def _():
        def bodyn(j, c):
            compute(j, xbf_ref[pl.ds(j * TB, TB), :])
            return c

        lax.fori_loop(0, nb, bodyn, 0)

    @pl.when(it == IT - 1)
    def _():
        @pl.when(nb >= 2)
        def _():
            wait_out(lax.rem(nb, 2))

        @pl.when(nb >= 1)
        def _():
            wait_out(lax.rem(nb - 1, 2))


def _ffn(meta, y, wg, wu, wd):
    grid_spec = pltpu.PrefetchScalarGridSpec(
        num_scalar_prefetch=1,
        grid=(E, IT),
        in_specs=[
            pl.BlockSpec(memory_space=pltpu.HBM),
            pl.BlockSpec((1, TI, H), lambda e, it, m: (e, it, 0)),
            pl.BlockSpec((1, TI, H), lambda e, it, m: (e, it, 0)),
            pl.BlockSpec((1, H, TI), lambda e, it, m: (e, 0, it)),
        ],
        out_specs=pl.BlockSpec(memory_space=pltpu.HBM),
        scratch_shapes=[
            pltpu.VMEM((_MAX_ROWS, H), jnp.bfloat16),
            pltpu.VMEM((_MAX_ROWS, H), jnp.float32),
            pltpu.VMEM((2, TB, H), jnp.float32),
            pltpu.VMEM((2, TB, H), jnp.float32),
            pltpu.SemaphoreType.DMA((2,)),
            pltpu.SemaphoreType.DMA((2,)),
        ],
    )
    return pl.pallas_call(
        _ffn_body,
        grid_spec=grid_spec,
        out_shape=jax.ShapeDtypeStruct((PADTOT, H), jnp.float32),
    )(meta, y, wg, wu, wd)


_CH = 16
_NCH = _TOK_PER_W // _CH


def _sc_combine_body(y_hbm, d0_hbm, d1_hbm, w0_hbm, w1_hbm, out_hbm,
                     i0_v, i1_v, w0_v, w1_v, a_v, b_v, o_v, gsa, gsb, osem):
    wid = lax.axis_index("s") * 2 + lax.axis_index("c")
    base = wid * _TOK_PER_W
    pltpu.sync_copy(d0_hbm.at[pl.ds(base, _TOK_PER_W)], i0_v)
    pltpu.sync_copy(d1_hbm.at[pl.ds(base, _TOK_PER_W)], i1_v)
    pltpu.sync_copy(w0_hbm.at[pl.ds(base, _TOK_PER_W)], w0_v)
    pltpu.sync_copy(w1_hbm.at[pl.ds(base, _TOK_PER_W)], w1_v)

    def gather(c, slot):
        return (
            pltpu.make_async_copy(y_hbm.at[i0_v.at[pl.ds(c * _CH, _CH)]],
                                  a_v.at[slot], gsa.at[slot]),
            pltpu.make_async_copy(y_hbm.at[i1_v.at[pl.ds(c * _CH, _CH)]],
                                  b_v.at[slot], gsb.at[slot]),
        )

    def writeback(c, slot):
        return pltpu.make_async_copy(
            o_v.at[slot], out_hbm.at[pl.ds(base + c * _CH, _CH)],
            osem.at[slot])

    ga, gb = gather(0, 0)
    ga.start()
    gb.start()
    for c in range(_NCH):
        slot = c % 2
        ga, gb = gather(c, slot)
        ga.wait()
        gb.wait()
        if c + 1 < _NCH:
            ga1, gb1 = gather(c + 1, 1 - slot)
            ga1.start()
            gb1.start()
        if c >= 2:
            writeback(c - 2, slot).wait()

        def row(r, _):
            s0 = w0_v[c * _CH + r, :]
            s1 = w1_v[c * _CH + r, :]

            def col(k, _):
                sl = pl.ds(k * 16, 16)
                o_v[slot, r, sl] = s0 * a_v[slot, r, sl] + s1 * b_v[slot, r, sl]
                return 0

            lax.fori_loop(0, H // 16, col, 0)
            return 0

        lax.fori_loop(0, _CH, row, 0)
        writeback(c, slot).start()
    writeback(_NCH - 2, 0).wait()
    writeback(_NCH - 1, 1).wait()


def _sc_combine(y, d0, d1, w0, w1):
    mesh = plsc.VectorSubcoreMesh(core_axis_name="c", subcore_axis_name="s")
    return pl.kernel(
        _sc_combine_body,
        out_type=jax.ShapeDtypeStruct((S, H), jnp.float32),
        mesh=mesh,
        scratch_types=[
            pltpu.VMEM((_TOK_PER_W,), jnp.int32),
            pltpu.VMEM((_TOK_PER_W,), jnp.int32),
            pltpu.VMEM((_TOK_PER_W, 16), jnp.float32),
            pltpu.VMEM((_TOK_PER_W, 16), jnp.float32),
            pltpu.VMEM((2, _CH, H), jnp.float32),
            pltpu.VMEM((2, _CH, H), jnp.float32),
            pltpu.VMEM((2, _CH, H), jnp.float32),
            pltpu.SemaphoreType.DMA((2,)),
            pltpu.SemaphoreType.DMA((2,)),
            pltpu.SemaphoreType.DMA((2,)),
        ],
    )(y, d0, d1, w0, w1)


def kernel(hidden_states, W_router, Wg, Wu, Wd):
    x = hidden_states.reshape(S, H)
    d0c, d1c, w0c, w1c, metac, auxc = _router(x, W_router)
    d0, d1 = d0c, d1c
    meta = metac.reshape(2 * E)
    y = _sc_dispatch(x, d0, d1)
    z = _ffn(meta, y, Wg, Wu, Wd)
    out = _sc_combine(z, d0, d1, w0c, w1c)
    return out.reshape(1, S, H), auxc.reshape(())
